# baseline (speedup 1.0000x reference)
import jax
import jax.numpy as jnp
from jax.experimental import pallas as pl
from jax.experimental.pallas import tpu as pltpu

_NC = 2
_NBJ = 8
_BR = 16384
_E = 16
_H = 32


def _reduce_kernel(x_ref, emb_ref, s1_ref, s2_ref, *, total_rows):
    i = pl.program_id(0)
    j = pl.program_id(1)
    blk = i * _NBJ + j
    xt = x_ref[...]
    eb = emb_ref[...]
    row = jax.lax.broadcasted_iota(jnp.int32, (_BR, 128), 0) + blk * _BR
    eb = jnp.where(row < total_rows, eb, 0.0)
    ki = jax.lax.broadcasted_iota(jnp.int32, (8, 128), 0)
    ci = jax.lax.broadcasted_iota(jnp.int32, (8, 128), 1)
    p = (ci // 16 == ki).astype(jnp.float32)
    xq = jax.lax.dot_general(xt, p, (((0,), (0,)), ((), ())),
                             preferred_element_type=jnp.float32)
    t = eb * xq
    p1 = jnp.sum(t, axis=0, keepdims=True)
    p2 = jnp.sum(t * t, axis=0, keepdims=True)

    @pl.when(j == 0)
    def _init():
        s1_ref[...] = p1[None]
        s2_ref[...] = p2[None]

    @pl.when(j != 0)
    def _acc():
        s1_ref[...] = s1_ref[...] + p1[None]
        s2_ref[...] = s2_ref[...] + p2[None]


def _head_kernel(s1_ref, s2_ref, wlog_ref, blog_ref, w1_ref, b1_ref,
                 w2_ref, b2_ref, wout_ref, bout_ref, o_ref):
    t1 = s1_ref[0] + s1_ref[1]
    t2 = s2_ref[0] + s2_ref[1]
    embf = t1[:, 0:_E]
    squ = t2[:, 0:_E]
    for k in range(1, 8):
        embf = embf + t1[:, k * _E:(k + 1) * _E]
        squ = squ + t2[:, k * _E:(k + 1) * _E]
    logistic = (jnp.sum(embf * wlog_ref[...], axis=1, keepdims=True)
                + blog_ref[...])
    fm = 0.5 * (embf * embf - squ)
    dn = (((1,), (1,)), ((), ()))
    h = jnp.maximum(jax.lax.dot_general(embf, w1_ref[...], dn,
                                        preferred_element_type=jnp.float32)
                    + b1_ref[...], 0.0)
    h = jnp.maximum(jax.lax.dot_general(h, w2_ref[...], dn,
                                        preferred_element_type=jnp.float32)
                    + b2_ref[...], 0.0)
    wout = wout_ref[...]
    z = (jnp.sum(h * wout[:, 0:_H], axis=1, keepdims=True)
         + jnp.sum(fm * wout[:, _H:_H + _E], axis=1, keepdims=True)
         + logistic * wout[:, _H + _E:_H + _E + 1]
         + bout_ref[...])
    o_ref[...] = jax.nn.sigmoid(z)


def kernel(x, emb, w_log, b_log, w1, b1, w2, b2, w_out, b_out):
    import functools
    f, e = emb.shape
    r = f // 8
    nblk = _NC * _NBJ
    emb2 = emb.reshape(r, 8 * e) + 0.0
    x8t = x.reshape(r, 8).T
    x8t = jnp.pad(x8t, ((0, 0), (0, nblk * _BR - r)))

    s1, s2 = pl.pallas_call(
        functools.partial(_reduce_kernel, total_rows=r),
        grid=(_NC, _NBJ),
        in_specs=[
            pl.BlockSpec((8, _BR), lambda i, j: (0, i * _NBJ + j)),
            pl.BlockSpec((_BR, 128), lambda i, j: (i * _NBJ + j, 0)),
        ],
        out_specs=[
            pl.BlockSpec((1, 1, 128), lambda i, j: (i, 0, 0)),
            pl.BlockSpec((1, 1, 128), lambda i, j: (i, 0, 0)),
        ],
        out_shape=[
            jax.ShapeDtypeStruct((_NC, 1, 128), jnp.float32),
            jax.ShapeDtypeStruct((_NC, 1, 128), jnp.float32),
        ],
        compiler_params=pltpu.CompilerParams(
            dimension_semantics=("parallel", "arbitrary")),
    )(x8t, emb2)

    out = pl.pallas_call(
        _head_kernel,
        out_shape=jax.ShapeDtypeStruct((1, 1), jnp.float32),
    )(s1, s2, w_log, b_log.reshape(1, 1), w1, b1.reshape(1, _H),
      w2, b2.reshape(1, _H), w_out, b_out.reshape(1, 1))
    return out.reshape(1)


# speedup vs baseline: 1.0775x; 1.0775x over previous
import jax
import jax.numpy as jnp
from jax.experimental import pallas as pl
from jax.experimental.pallas import tpu as pltpu

_NC = 2
_NBJ = 25
_BR = 5000
_E = 16
_H = 32


def _reduce_kernel(x_ref, emb_ref, s1_ref, s2_ref):
    j = pl.program_id(1)
    xt = x_ref[0]
    eb = emb_ref[...]
    ki = jax.lax.broadcasted_iota(jnp.int32, (8, 128), 0)
    ci = jax.lax.broadcasted_iota(jnp.int32, (8, 128), 1)
    p = (ci // 16 == ki).astype(jnp.float32)
    xq = jax.lax.dot_general(xt, p, (((0,), (0,)), ((), ())),
                             preferred_element_type=jnp.float32)
    t = eb * xq
    p1 = jnp.sum(t, axis=0, keepdims=True)
    p2 = jnp.sum(t * t, axis=0, keepdims=True)

    @pl.when(j == 0)
    def _init():
        s1_ref[...] = p1[None]
        s2_ref[...] = p2[None]

    @pl.when(j != 0)
    def _acc():
        s1_ref[...] = s1_ref[...] + p1[None]
        s2_ref[...] = s2_ref[...] + p2[None]


def _head_kernel(s1_ref, s2_ref, wlog_ref, blog_ref, w1_ref, b1_ref,
                 w2_ref, b2_ref, wout_ref, bout_ref, o_ref):
    t1 = s1_ref[0] + s1_ref[1]
    t2 = s2_ref[0] + s2_ref[1]
    embf = t1[:, 0:_E]
    squ = t2[:, 0:_E]
    for k in range(1, 8):
        embf = embf + t1[:, k * _E:(k + 1) * _E]
        squ = squ + t2[:, k * _E:(k + 1) * _E]
    logistic = (jnp.sum(embf * wlog_ref[...], axis=1, keepdims=True)
                + blog_ref[...])
    fm = 0.5 * (embf * embf - squ)
    dn = (((1,), (1,)), ((), ()))
    h = jnp.maximum(jax.lax.dot_general(embf, w1_ref[...], dn,
                                        preferred_element_type=jnp.float32)
                    + b1_ref[...], 0.0)
    h = jnp.maximum(jax.lax.dot_general(h, w2_ref[...], dn,
                                        preferred_element_type=jnp.float32)
                    + b2_ref[...], 0.0)
    wout = wout_ref[...]
    z = (jnp.sum(h * wout[:, 0:_H], axis=1, keepdims=True)
         + jnp.sum(fm * wout[:, _H:_H + _E], axis=1, keepdims=True)
         + logistic * wout[:, _H + _E:_H + _E + 1]
         + bout_ref[...])
    o_ref[...] = jax.nn.sigmoid(z)


def kernel(x, emb, w_log, b_log, w1, b1, w2, b2, w_out, b_out):
    f, e = emb.shape
    r = f // 8
    nblk = _NC * _NBJ
    emb2 = emb.reshape(r, 8 * e)
    x3t = x.reshape(nblk, _BR, 8).transpose(0, 2, 1)

    s1, s2 = pl.pallas_call(
        _reduce_kernel,
        grid=(_NC, _NBJ),
        in_specs=[
            pl.BlockSpec((1, 8, _BR), lambda i, j: (i * _NBJ + j, 0, 0)),
            pl.BlockSpec((_BR, 128), lambda i, j: (i * _NBJ + j, 0)),
        ],
        out_specs=[
            pl.BlockSpec((1, 1, 128), lambda i, j: (i, 0, 0)),
            pl.BlockSpec((1, 1, 128), lambda i, j: (i, 0, 0)),
        ],
        out_shape=[
            jax.ShapeDtypeStruct((_NC, 1, 128), jnp.float32),
            jax.ShapeDtypeStruct((_NC, 1, 128), jnp.float32),
        ],
        compiler_params=pltpu.CompilerParams(
            dimension_semantics=("parallel", "arbitrary")),
    )(x3t, emb2)

    out = pl.pallas_call(
        _head_kernel,
        out_shape=jax.ShapeDtypeStruct((1, 1), jnp.float32),
    )(s1, s2, w_log, b_log.reshape(1, 1), w1, b1.reshape(1, _H),
      w2, b2.reshape(1, _H), w_out, b_out.reshape(1, 1))
    return out.reshape(1)


# speedup vs baseline: 1.2597x; 1.1691x over previous
import functools

import jax
import jax.numpy as jnp
from jax.experimental import pallas as pl
from jax.experimental.pallas import tpu as pltpu

_NC = 2
_NBJ = 25
_Q = 10
_E = 16
_H = 32


def _reduce_kernel(*refs, nq):
    x_refs = refs[:nq]
    e_refs = refs[nq:2 * nq]
    s1_ref, s2_ref = refs[2 * nq:]
    j = pl.program_id(1)
    dn = (((1,), (0,)), ((), ()))
    p1 = None
    p2 = None
    for q in range(nq):
        xr = x_refs[q][0]
        eb = e_refs[q][...]
        a = jax.lax.dot_general(xr, eb, dn,
                                preferred_element_type=jnp.float32)
        b = jax.lax.dot_general(xr * xr, eb * eb, dn,
                                preferred_element_type=jnp.float32)
        p1 = a if p1 is None else p1 + a
        p2 = b if p2 is None else p2 + b

    @pl.when(j == 0)
    def _init():
        s1_ref[...] = p1[None]
        s2_ref[...] = p2[None]

    @pl.when(j != 0)
    def _acc():
        s1_ref[...] = s1_ref[...] + p1[None]
        s2_ref[...] = s2_ref[...] + p2[None]


def _head_kernel(s1_ref, s2_ref, wlog_ref, blog_ref, w1_ref, b1_ref,
                 w2_ref, b2_ref, wout_ref, bout_ref, o_ref):
    embf = s1_ref[0] + s1_ref[1]
    squ = s2_ref[0] + s2_ref[1]
    logistic = (jnp.sum(embf * wlog_ref[...], axis=1, keepdims=True)
                + blog_ref[...])
    fm = 0.5 * (embf * embf - squ)
    dn = (((1,), (1,)), ((), ()))
    h = jnp.maximum(jax.lax.dot_general(embf, w1_ref[...], dn,
                                        preferred_element_type=jnp.float32)
                    + b1_ref[...], 0.0)
    h = jnp.maximum(jax.lax.dot_general(h, w2_ref[...], dn,
                                        preferred_element_type=jnp.float32)
                    + b2_ref[...], 0.0)
    wout = wout_ref[...]
    z = (jnp.sum(h * wout[:, 0:_H], axis=1, keepdims=True)
         + jnp.sum(fm * wout[:, _H:_H + _E], axis=1, keepdims=True)
         + logistic * wout[:, _H + _E:_H + _E + 1]
         + bout_ref[...])
    o_ref[...] = jax.nn.sigmoid(z)


def kernel(x, emb, w_log, b_log, w1, b1, w2, b2, w_out, b_out):
    f, e = emb.shape
    nblk = _NC * _NBJ * _Q
    bf = f // nblk
    x3 = x.reshape(nblk, 1, bf)

    def x_map(q):
        return lambda i, j: ((i * _NBJ + j) * _Q + q, 0, 0)

    def e_map(q):
        return lambda i, j: ((i * _NBJ + j) * _Q + q, 0)

    in_specs = (
        [pl.BlockSpec((1, 1, bf), x_map(q)) for q in range(_Q)]
        + [pl.BlockSpec((bf, e), e_map(q)) for q in range(_Q)]
    )

    s1, s2 = pl.pallas_call(
        functools.partial(_reduce_kernel, nq=_Q),
        grid=(_NC, _NBJ),
        in_specs=in_specs,
        out_specs=[
            pl.BlockSpec((1, 1, e), lambda i, j: (i, 0, 0)),
            pl.BlockSpec((1, 1, e), lambda i, j: (i, 0, 0)),
        ],
        out_shape=[
            jax.ShapeDtypeStruct((_NC, 1, e), jnp.float32),
            jax.ShapeDtypeStruct((_NC, 1, e), jnp.float32),
        ],
        compiler_params=pltpu.CompilerParams(
            dimension_semantics=("parallel", "arbitrary")),
    )(*([x3] * _Q), *([emb] * _Q))

    out = pl.pallas_call(
        _head_kernel,
        out_shape=jax.ShapeDtypeStruct((1, 1), jnp.float32),
    )(s1, s2, w_log, b_log.reshape(1, 1), w1, b1.reshape(1, _H),
      w2, b2.reshape(1, _H), w_out, b_out.reshape(1, 1))
    return out.reshape(1)


# speedup vs baseline: 1.2614x; 1.0013x over previous
import functools

import jax
import jax.numpy as jnp
from jax.experimental import pallas as pl
from jax.experimental.pallas import tpu as pltpu

_NC = 2
_NBJ = 25
_Q = 5
_E = 16
_H = 32


def _reduce_kernel(*refs, nq):
    x_refs = refs[:nq]
    e_refs = refs[nq:2 * nq]
    s1_ref, s2_ref = refs[2 * nq:]
    j = pl.program_id(1)
    dn = (((1,), (0,)), ((), ()))
    p1 = None
    p2 = None
    for q in range(nq):
        xr = x_refs[q][0]
        eb = e_refs[q][...]
        a = jax.lax.dot_general(xr, eb, dn,
                                preferred_element_type=jnp.float32)
        b = jax.lax.dot_general(xr * xr, eb * eb, dn,
                                preferred_element_type=jnp.float32)
        p1 = a if p1 is None else p1 + a
        p2 = b if p2 is None else p2 + b

    @pl.when(j == 0)
    def _init():
        s1_ref[...] = p1[None]
        s2_ref[...] = p2[None]

    @pl.when(j != 0)
    def _acc():
        s1_ref[...] = s1_ref[...] + p1[None]
        s2_ref[...] = s2_ref[...] + p2[None]


def _head_kernel(s1_ref, s2_ref, wlog_ref, blog_ref, w1_ref, b1_ref,
                 w2_ref, b2_ref, wout_ref, bout_ref, o_ref):
    embf = s1_ref[0] + s1_ref[1]
    squ = s2_ref[0] + s2_ref[1]
    logistic = (jnp.sum(embf * wlog_ref[...], axis=1, keepdims=True)
                + blog_ref[...])
    fm = 0.5 * (embf * embf - squ)
    dn = (((1,), (1,)), ((), ()))
    h = jnp.maximum(jax.lax.dot_general(embf, w1_ref[...], dn,
                                        preferred_element_type=jnp.float32)
                    + b1_ref[...], 0.0)
    h = jnp.maximum(jax.lax.dot_general(h, w2_ref[...], dn,
                                        preferred_element_type=jnp.float32)
                    + b2_ref[...], 0.0)
    wout = wout_ref[...]
    z = (jnp.sum(h * wout[:, 0:_H], axis=1, keepdims=True)
         + jnp.sum(fm * wout[:, _H:_H + _E], axis=1, keepdims=True)
         + logistic * wout[:, _H + _E:_H + _E + 1]
         + bout_ref[...])
    o_ref[...] = jax.nn.sigmoid(z)


def kernel(x, emb, w_log, b_log, w1, b1, w2, b2, w_out, b_out):
    f, e = emb.shape
    nblk = _NC * _NBJ * _Q
    bf = f // nblk
    x3 = x.reshape(nblk, 1, bf)

    def x_map(q):
        return lambda i, j: ((i * _NBJ + j) * _Q + q, 0, 0)

    def e_map(q):
        return lambda i, j: ((i * _NBJ + j) * _Q + q, 0)

    in_specs = (
        [pl.BlockSpec((1, 1, bf), x_map(q)) for q in range(_Q)]
        + [pl.BlockSpec((bf, e), e_map(q)) for q in range(_Q)]
    )

    s1, s2 = pl.pallas_call(
        functools.partial(_reduce_kernel, nq=_Q),
        grid=(_NC, _NBJ),
        in_specs=in_specs,
        out_specs=[
            pl.BlockSpec((1, 1, e), lambda i, j: (i, 0, 0)),
            pl.BlockSpec((1, 1, e), lambda i, j: (i, 0, 0)),
        ],
        out_shape=[
            jax.ShapeDtypeStruct((_NC, 1, e), jnp.float32),
            jax.ShapeDtypeStruct((_NC, 1, e), jnp.float32),
        ],
        compiler_params=pltpu.CompilerParams(
            dimension_semantics=("parallel", "arbitrary")),
    )(*([x3] * _Q), *([emb] * _Q))

    out = pl.pallas_call(
        _head_kernel,
        out_shape=jax.ShapeDtypeStruct((1, 1), jnp.float32),
    )(s1, s2, w_log, b_log.reshape(1, 1), w1, b1.reshape(1, _H),
      w2, b2.reshape(1, _H), w_out, b_out.reshape(1, 1))
    return out.reshape(1)
